# baseline (speedup 1.0000x reference)
import functools

import jax
import jax.numpy as jnp
from jax import lax
from jax.experimental import pallas as pl
from jax.experimental.pallas import tpu as pltpu
from jax.experimental.pallas import tpu_sc as plsc

VOCAB = 100000
EMBED = 64
N = 16384

NUM_CORES = 2
NUM_SUBCORES = 16
NW = NUM_CORES * NUM_SUBCORES
CHUNK = N // NW
LANES = 16


def _gather_sum_body(idx_hbm, emb_hbm, out_hbm, idx_v, rows_v, acc_v, sem):
    wid = lax.axis_index("s") * NUM_CORES + lax.axis_index("c")
    base = wid * CHUNK
    pltpu.sync_copy(idx_hbm.at[pl.ds(base, CHUNK)], idx_v)
    pltpu.async_copy(emb_hbm.at[idx_v], rows_v, sem).wait()

    nvec = EMBED // LANES

    def body(r, carry):
        return tuple(
            carry[c] + rows_v[r, pl.ds(c * LANES, LANES)] for c in range(nvec)
        )

    init = tuple(jnp.zeros((LANES,), jnp.float32) for _ in range(nvec))
    acc = lax.fori_loop(0, CHUNK, body, init, unroll=8)
    for c in range(nvec):
        acc_v[pl.ds(c * LANES, LANES)] = acc[c]
    pltpu.sync_copy(acc_v, out_hbm.at[wid])


@functools.cache
def _gather_sum():
    return pl.kernel(
        _gather_sum_body,
        out_type=jax.ShapeDtypeStruct((NW, EMBED), jnp.float32),
        mesh=plsc.VectorSubcoreMesh(
            core_axis_name="c", subcore_axis_name="s",
            num_cores=NUM_CORES, num_subcores=NUM_SUBCORES,
        ),
        scratch_types=[
            pltpu.VMEM((CHUNK,), jnp.int32),
            pltpu.VMEM((CHUNK, EMBED), jnp.float32),
            pltpu.VMEM((EMBED,), jnp.float32),
            pltpu.SemaphoreType.DMA,
        ],
        compiler_params=pltpu.CompilerParams(use_tc_tiling_on_sc=False),
    )


BV = 8192


def _project_body(p_ref, w_ref, b_ref, o_ref):
    e = jnp.sum(p_ref[...], axis=0, keepdims=True)
    acc = lax.dot_general(
        e, w_ref[...], (((1,), (1,)), ((), ())),
        preferred_element_type=jnp.float32,
    )
    o_ref[...] = acc[0] + b_ref[...]


def _project(partials, W, b):
    grid = (VOCAB + BV - 1) // BV
    return pl.pallas_call(
        _project_body,
        grid=(grid,),
        in_specs=[
            pl.BlockSpec((NW, EMBED), lambda i: (0, 0)),
            pl.BlockSpec((BV, EMBED), lambda i: (i, 0)),
            pl.BlockSpec((BV,), lambda i: (i,)),
        ],
        out_specs=pl.BlockSpec((BV,), lambda i: (i,)),
        out_shape=jax.ShapeDtypeStruct((VOCAB,), jnp.float32),
    )(partials, W, b)


def kernel(inputs, emb, W, b):
    idx = inputs.astype(jnp.int32)
    partials = _gather_sum()(idx, emb)
    return _project(partials, W, b)


# speedup vs baseline: 2.7579x; 2.7579x over previous
import functools

import jax
import jax.numpy as jnp
from jax import lax
from jax.experimental import pallas as pl
from jax.experimental.pallas import tpu as pltpu
from jax.experimental.pallas import tpu_sc as plsc

VOCAB = 100000
EMBED = 64
N = 16384

NUM_CORES = 2
NUM_SUBCORES = 16
LANES = 16

CB = 128
NROWS = N // CB
ROWS_PER_TILE = NROWS // (NUM_CORES * NUM_SUBCORES)

CHUNK = 6256
LAST_CHUNK = VOCAB - CHUNK * (NUM_SUBCORES - 1)


def _count_body(idx_hbm, out_hbm, idx_v, ones_v, zeros_v, counts_sh):
    cc = lax.axis_index("c")
    s = lax.axis_index("s")

    one = jnp.full((LANES,), 1.0, jnp.float32)
    for k in range(CB // LANES):
        ones_v[pl.ds(k * LANES, LANES)] = one

    zero = jnp.zeros((LANES,), jnp.float32)

    def zfill(i, carry):
        zeros_v[pl.ds(pl.multiple_of(i * LANES, LANES), LANES)] = zero
        return carry

    lax.fori_loop(0, CHUNK // LANES, zfill, 0)

    off = pl.multiple_of(s * CHUNK, 8)

    @pl.when(s < NUM_SUBCORES - 1)
    def _():
        pltpu.sync_copy(zeros_v, counts_sh.at[pl.ds(off, CHUNK)])

    @pl.when(s == NUM_SUBCORES - 1)
    def _():
        pltpu.sync_copy(zeros_v.at[pl.ds(0, LAST_CHUNK)],
                        counts_sh.at[pl.ds(off, LAST_CHUNK)])

    plsc.subcore_barrier()

    for j in range(ROWS_PER_TILE):
        row = cc * (NROWS // NUM_CORES) + s * ROWS_PER_TILE + j
        pltpu.sync_copy(idx_hbm.at[row], idx_v)
        pltpu.sync_copy(ones_v, counts_sh.at[idx_v], add=True)

    plsc.subcore_barrier()

    @pl.when(s < NUM_SUBCORES - 1)
    def _():
        pltpu.sync_copy(counts_sh.at[pl.ds(off, CHUNK)],
                        out_hbm.at[cc, pl.ds(off, CHUNK)])

    @pl.when(s == NUM_SUBCORES - 1)
    def _():
        pltpu.sync_copy(counts_sh.at[pl.ds(off, LAST_CHUNK)],
                        out_hbm.at[cc, pl.ds(off, LAST_CHUNK)])


@functools.cache
def _count_kernel():
    return pl.kernel(
        _count_body,
        out_type=jax.ShapeDtypeStruct((NUM_CORES, VOCAB), jnp.float32),
        mesh=plsc.VectorSubcoreMesh(
            core_axis_name="c", subcore_axis_name="s",
            num_cores=NUM_CORES, num_subcores=NUM_SUBCORES,
        ),
        scratch_types=[
            pltpu.VMEM((CB,), jnp.int32),
            pltpu.VMEM((CB,), jnp.float32),
            pltpu.VMEM((CHUNK,), jnp.float32),
            pltpu.VMEM_SHARED((VOCAB,), jnp.float32),
        ],
        compiler_params=pltpu.CompilerParams(use_tc_tiling_on_sc=False),
    )


BV = 16384


def _accum_body(c_ref, et_ref, acc_ref):
    i = pl.program_id(0)

    @pl.when(i == 0)
    def _():
        acc_ref[...] = jnp.zeros_like(acc_ref)

    col = i * BV + lax.broadcasted_iota(jnp.int32, (1, BV), 1)
    valid = col < VOCAB
    c = jnp.where(jnp.broadcast_to(valid, (NUM_CORES, BV)), c_ref[...], 0.0)
    et = jnp.where(jnp.broadcast_to(valid, (EMBED, BV)), et_ref[...], 0.0)
    acc_ref[...] += lax.dot_general(
        et, c, (((1,), (1,)), ((), ())),
        preferred_element_type=jnp.float32,
    )


def _accum(counts, embT):
    grid = (VOCAB + BV - 1) // BV
    return pl.pallas_call(
        _accum_body,
        grid=(grid,),
        in_specs=[
            pl.BlockSpec((NUM_CORES, BV), lambda i: (0, i)),
            pl.BlockSpec((EMBED, BV), lambda i: (0, i)),
        ],
        out_specs=pl.BlockSpec((EMBED, NUM_CORES), lambda i: (0, 0)),
        out_shape=jax.ShapeDtypeStruct((EMBED, NUM_CORES), jnp.float32),
    )(counts, embT)


def _project_body(a_ref, wt_ref, b_ref, o_ref):
    e = jnp.sum(a_ref[...], axis=1)
    res = lax.dot_general(
        e[None, :], wt_ref[...], (((1,), (0,)), ((), ())),
        preferred_element_type=jnp.float32,
    )
    o_ref[...] = res[0] + b_ref[...]


def _project(acc, WT, b):
    grid = (VOCAB + BV - 1) // BV
    return pl.pallas_call(
        _project_body,
        grid=(grid,),
        in_specs=[
            pl.BlockSpec((EMBED, NUM_CORES), lambda i: (0, 0)),
            pl.BlockSpec((EMBED, BV), lambda i: (0, i)),
            pl.BlockSpec((BV,), lambda i: (i,)),
        ],
        out_specs=pl.BlockSpec((BV,), lambda i: (i,)),
        out_shape=jax.ShapeDtypeStruct((VOCAB,), jnp.float32),
    )(acc, WT, b)


def kernel(inputs, emb, W, b):
    idx2 = inputs.astype(jnp.int32).reshape(NROWS, CB)
    counts = _count_kernel()(idx2)
    acc = _accum(counts, emb.T)
    return _project(acc, W.T, b)


# speedup vs baseline: 3.0446x; 1.1039x over previous
import functools

import jax
import jax.numpy as jnp
from jax import lax
from jax.experimental import pallas as pl
from jax.experimental.pallas import tpu as pltpu
from jax.experimental.pallas import tpu_sc as plsc

VOCAB = 100000
EMBED = 64
N = 16384

NUM_CORES = 2
NUM_SUBCORES = 16
LANES = 16

CB = 128
NROWS = N // CB
ROWS_PER_TILE = NROWS // (NUM_CORES * NUM_SUBCORES)

CHUNK = 6256
LAST_CHUNK = VOCAB - CHUNK * (NUM_SUBCORES - 1)


def _count_body(idx_hbm, out0_hbm, out1_hbm, idx_v, ones_v, zeros_v, counts_sh):
    cc = lax.axis_index("c")
    s = lax.axis_index("s")

    one = jnp.full((LANES,), 1.0, jnp.float32)
    for k in range(CB // LANES):
        ones_v[pl.ds(k * LANES, LANES)] = one

    zero = jnp.zeros((LANES,), jnp.float32)

    def zfill(i, carry):
        zeros_v[pl.ds(pl.multiple_of(i * LANES, LANES), LANES)] = zero
        return carry

    lax.fori_loop(0, CHUNK // LANES, zfill, 0)

    off = pl.multiple_of(s * CHUNK, 8)

    @pl.when(s < NUM_SUBCORES - 1)
    def _():
        pltpu.sync_copy(zeros_v, counts_sh.at[pl.ds(off, CHUNK)])

    @pl.when(s == NUM_SUBCORES - 1)
    def _():
        pltpu.sync_copy(zeros_v.at[pl.ds(0, LAST_CHUNK)],
                        counts_sh.at[pl.ds(off, LAST_CHUNK)])

    plsc.subcore_barrier()

    for j in range(ROWS_PER_TILE):
        row = cc * (NROWS // NUM_CORES) + s * ROWS_PER_TILE + j
        pltpu.sync_copy(idx_hbm.at[row], idx_v)
        pltpu.sync_copy(ones_v, counts_sh.at[idx_v], add=True)

    plsc.subcore_barrier()

    @pl.when(jnp.logical_and(s < NUM_SUBCORES - 1, cc == 0))
    def _():
        pltpu.sync_copy(counts_sh.at[pl.ds(off, CHUNK)],
                        out0_hbm.at[pl.ds(off, CHUNK)])

    @pl.when(jnp.logical_and(s == NUM_SUBCORES - 1, cc == 0))
    def _():
        pltpu.sync_copy(counts_sh.at[pl.ds(off, LAST_CHUNK)],
                        out0_hbm.at[pl.ds(off, LAST_CHUNK)])

    @pl.when(jnp.logical_and(s < NUM_SUBCORES - 1, cc == 1))
    def _():
        pltpu.sync_copy(counts_sh.at[pl.ds(off, CHUNK)],
                        out1_hbm.at[pl.ds(off, CHUNK)])

    @pl.when(jnp.logical_and(s == NUM_SUBCORES - 1, cc == 1))
    def _():
        pltpu.sync_copy(counts_sh.at[pl.ds(off, LAST_CHUNK)],
                        out1_hbm.at[pl.ds(off, LAST_CHUNK)])


@functools.cache
def _count_kernel():
    return pl.kernel(
        _count_body,
        out_type=[
            jax.ShapeDtypeStruct((VOCAB,), jnp.float32),
            jax.ShapeDtypeStruct((VOCAB,), jnp.float32),
        ],
        mesh=plsc.VectorSubcoreMesh(
            core_axis_name="c", subcore_axis_name="s",
            num_cores=NUM_CORES, num_subcores=NUM_SUBCORES,
        ),
        scratch_types=[
            pltpu.VMEM((CB,), jnp.int32),
            pltpu.VMEM((CB,), jnp.float32),
            pltpu.VMEM((CHUNK,), jnp.float32),
            pltpu.VMEM_SHARED((VOCAB,), jnp.float32),
        ],
        compiler_params=pltpu.CompilerParams(use_tc_tiling_on_sc=False),
    )


BV = 20480


def _accum_body(c0_ref, c1_ref, et_ref, acc_ref):
    i = pl.program_id(0)

    @pl.when(i == 0)
    def _():
        acc_ref[...] = jnp.zeros_like(acc_ref)

    col = i * BV + lax.broadcasted_iota(jnp.int32, (1, BV), 1)
    valid = col < VOCAB
    c = jnp.where(valid, (c0_ref[...] + c1_ref[...])[None, :], 0.0)
    et = jnp.where(jnp.broadcast_to(valid, (EMBED, BV)), et_ref[...], 0.0)
    acc_ref[...] += lax.dot_general(
        c, et, (((1,), (1,)), ((), ())),
        preferred_element_type=jnp.float32,
    )


def _accum(c0, c1, embT):
    grid = (VOCAB + BV - 1) // BV
    return pl.pallas_call(
        _accum_body,
        grid=(grid,),
        in_specs=[
            pl.BlockSpec((BV,), lambda i: (i,)),
            pl.BlockSpec((BV,), lambda i: (i,)),
            pl.BlockSpec((EMBED, BV), lambda i: (0, i)),
        ],
        out_specs=pl.BlockSpec((1, EMBED), lambda i: (0, 0)),
        out_shape=jax.ShapeDtypeStruct((1, EMBED), jnp.float32),
    )(c0, c1, embT)


def _project_body(a_ref, wt_ref, b_ref, o_ref):
    res = lax.dot_general(
        a_ref[...], wt_ref[...], (((1,), (0,)), ((), ())),
        preferred_element_type=jnp.float32,
    )
    o_ref[...] = res[0] + b_ref[...]


def _project(acc, WT, b):
    grid = (VOCAB + BV - 1) // BV
    return pl.pallas_call(
        _project_body,
        grid=(grid,),
        in_specs=[
            pl.BlockSpec((1, EMBED), lambda i: (0, 0)),
            pl.BlockSpec((EMBED, BV), lambda i: (0, i)),
            pl.BlockSpec((BV,), lambda i: (i,)),
        ],
        out_specs=pl.BlockSpec((BV,), lambda i: (i,)),
        out_shape=jax.ShapeDtypeStruct((VOCAB,), jnp.float32),
    )(acc, WT, b)


def kernel(inputs, emb, W, b):
    idx2 = inputs.astype(jnp.int32).reshape(NROWS, CB)
    c0, c1 = _count_kernel()(idx2)
    acc = _accum(c0, c1, emb.T)
    return _project(acc, W.T, b)


# speedup vs baseline: 3.2839x; 1.0786x over previous
import functools

import jax
import jax.numpy as jnp
from jax import lax
from jax.experimental import pallas as pl
from jax.experimental.pallas import tpu as pltpu
from jax.experimental.pallas import tpu_sc as plsc

VOCAB = 100000
EMBED = 64
N = 16384

NUM_CORES = 2
NUM_SUBCORES = 16
LANES = 16

CB = 128
NROWS = N // CB
ROWS_PER_TILE = NROWS // (NUM_CORES * NUM_SUBCORES)

CHUNK = 6256
LAST_CHUNK = VOCAB - CHUNK * (NUM_SUBCORES - 1)


def _count_body(idx_hbm, out0_hbm, out1_hbm, idx_v, ones_v, zeros_v, counts_sh,
                sem):
    cc = lax.axis_index("c")
    s = lax.axis_index("s")

    row0 = cc * (NROWS // NUM_CORES) + s * ROWS_PER_TILE
    idx_cps = [
        pltpu.async_copy(idx_hbm.at[row0 + j], idx_v.at[j], sem)
        for j in range(ROWS_PER_TILE)
    ]

    one = jnp.full((LANES,), 1.0, jnp.float32)
    for k in range(CB // LANES):
        ones_v[pl.ds(k * LANES, LANES)] = one

    zero = jnp.zeros((LANES,), jnp.float32)

    def zfill(i, carry):
        zeros_v[pl.ds(pl.multiple_of(i * LANES, LANES), LANES)] = zero
        return carry

    lax.fori_loop(0, CHUNK // LANES, zfill, 0, unroll=8)

    off = pl.multiple_of(s * CHUNK, 8)

    @pl.when(s < NUM_SUBCORES - 1)
    def _():
        pltpu.sync_copy(zeros_v, counts_sh.at[pl.ds(off, CHUNK)])

    @pl.when(s == NUM_SUBCORES - 1)
    def _():
        pltpu.sync_copy(zeros_v.at[pl.ds(0, LAST_CHUNK)],
                        counts_sh.at[pl.ds(off, LAST_CHUNK)])

    for cp in idx_cps:
        cp.wait()
    plsc.subcore_barrier()

    for j in range(ROWS_PER_TILE):
        pltpu.sync_copy(ones_v, counts_sh.at[idx_v.at[j]], add=True)

    plsc.subcore_barrier()

    @pl.when(jnp.logical_and(s < NUM_SUBCORES - 1, cc == 0))
    def _():
        pltpu.sync_copy(counts_sh.at[pl.ds(off, CHUNK)],
                        out0_hbm.at[pl.ds(off, CHUNK)])

    @pl.when(jnp.logical_and(s == NUM_SUBCORES - 1, cc == 0))
    def _():
        pltpu.sync_copy(counts_sh.at[pl.ds(off, LAST_CHUNK)],
                        out0_hbm.at[pl.ds(off, LAST_CHUNK)])

    @pl.when(jnp.logical_and(s < NUM_SUBCORES - 1, cc == 1))
    def _():
        pltpu.sync_copy(counts_sh.at[pl.ds(off, CHUNK)],
                        out1_hbm.at[pl.ds(off, CHUNK)])

    @pl.when(jnp.logical_and(s == NUM_SUBCORES - 1, cc == 1))
    def _():
        pltpu.sync_copy(counts_sh.at[pl.ds(off, LAST_CHUNK)],
                        out1_hbm.at[pl.ds(off, LAST_CHUNK)])


@functools.cache
def _count_kernel():
    return pl.kernel(
        _count_body,
        out_type=[
            jax.ShapeDtypeStruct((VOCAB,), jnp.float32),
            jax.ShapeDtypeStruct((VOCAB,), jnp.float32),
        ],
        mesh=plsc.VectorSubcoreMesh(
            core_axis_name="c", subcore_axis_name="s",
            num_cores=NUM_CORES, num_subcores=NUM_SUBCORES,
        ),
        scratch_types=[
            pltpu.VMEM((ROWS_PER_TILE, CB), jnp.int32),
            pltpu.VMEM((CB,), jnp.float32),
            pltpu.VMEM((CHUNK,), jnp.float32),
            pltpu.VMEM_SHARED((VOCAB,), jnp.float32),
            pltpu.SemaphoreType.DMA,
        ],
        compiler_params=pltpu.CompilerParams(use_tc_tiling_on_sc=False),
    )


BV = 20480


def _accum_body(c0_ref, c1_ref, et_ref, acc_ref):
    i = pl.program_id(0)

    @pl.when(i == 0)
    def _():
        acc_ref[...] = jnp.zeros_like(acc_ref)

    col = i * BV + lax.broadcasted_iota(jnp.int32, (1, BV), 1)
    valid = col < VOCAB
    c = jnp.where(valid, (c0_ref[...] + c1_ref[...])[None, :], 0.0)
    et = jnp.where(jnp.broadcast_to(valid, (EMBED, BV)), et_ref[...], 0.0)
    acc_ref[...] += lax.dot_general(
        c, et, (((1,), (1,)), ((), ())),
        preferred_element_type=jnp.float32,
    )


def _accum(c0, c1, embT):
    grid = (VOCAB + BV - 1) // BV
    return pl.pallas_call(
        _accum_body,
        grid=(grid,),
        in_specs=[
            pl.BlockSpec((BV,), lambda i: (i,)),
            pl.BlockSpec((BV,), lambda i: (i,)),
            pl.BlockSpec((EMBED, BV), lambda i: (0, i)),
        ],
        out_specs=pl.BlockSpec((1, EMBED), lambda i: (0, 0)),
        out_shape=jax.ShapeDtypeStruct((1, EMBED), jnp.float32),
    )(c0, c1, embT)


def _project_body(a_ref, wt_ref, b_ref, o_ref):
    res = lax.dot_general(
        a_ref[...], wt_ref[...], (((1,), (0,)), ((), ())),
        preferred_element_type=jnp.float32,
    )
    o_ref[...] = res[0] + b_ref[...]


def _project(acc, WT, b):
    grid = (VOCAB + BV - 1) // BV
    return pl.pallas_call(
        _project_body,
        grid=(grid,),
        in_specs=[
            pl.BlockSpec((1, EMBED), lambda i: (0, 0)),
            pl.BlockSpec((EMBED, BV), lambda i: (0, i)),
            pl.BlockSpec((BV,), lambda i: (i,)),
        ],
        out_specs=pl.BlockSpec((BV,), lambda i: (i,)),
        out_shape=jax.ShapeDtypeStruct((VOCAB,), jnp.float32),
    )(acc, WT, b)


def kernel(inputs, emb, W, b):
    idx2 = inputs.astype(jnp.int32).reshape(NROWS, CB)
    c0, c1 = _count_kernel()(idx2)
    acc = _accum(c0, c1, emb.T)
    return _project(acc, W.T, b)


# speedup vs baseline: 3.3111x; 1.0083x over previous
import functools

import jax
import jax.numpy as jnp
from jax import lax
from jax.experimental import pallas as pl
from jax.experimental.pallas import tpu as pltpu
from jax.experimental.pallas import tpu_sc as plsc

VOCAB = 100000
EMBED = 64
N = 16384

NUM_CORES = 2
NUM_SUBCORES = 16
LANES = 16

CB = 128
NROWS = N // CB
ROWS_PER_TILE = NROWS // (NUM_CORES * NUM_SUBCORES)

CHUNK = 6256
LAST_CHUNK = VOCAB - CHUNK * (NUM_SUBCORES - 1)


def _count_body(idx_hbm, out0_hbm, out1_hbm, idx_v, ones_v, zeros_v, counts_sh,
                sem):
    cc = lax.axis_index("c")
    s = lax.axis_index("s")

    row0 = cc * (NROWS // NUM_CORES) + s * ROWS_PER_TILE
    idx_cps = [
        pltpu.async_copy(idx_hbm.at[row0 + j], idx_v.at[j], sem)
        for j in range(ROWS_PER_TILE)
    ]

    one = jnp.full((LANES,), 1.0, jnp.float32)
    for k in range(CB // LANES):
        ones_v[pl.ds(k * LANES, LANES)] = one

    zero = jnp.zeros((LANES,), jnp.float32)

    def zfill(i, carry):
        zeros_v[pl.ds(pl.multiple_of(i * LANES, LANES), LANES)] = zero
        return carry

    lax.fori_loop(0, CHUNK // LANES, zfill, 0, unroll=8)

    off = pl.multiple_of(s * CHUNK, 8)

    @pl.when(s < NUM_SUBCORES - 1)
    def _():
        pltpu.sync_copy(zeros_v, counts_sh.at[pl.ds(off, CHUNK)])

    @pl.when(s == NUM_SUBCORES - 1)
    def _():
        pltpu.sync_copy(zeros_v.at[pl.ds(0, LAST_CHUNK)],
                        counts_sh.at[pl.ds(off, LAST_CHUNK)])

    for cp in idx_cps:
        cp.wait()
    plsc.subcore_barrier()

    for j in range(ROWS_PER_TILE):
        pltpu.sync_copy(ones_v, counts_sh.at[idx_v.at[j]], add=True)

    plsc.subcore_barrier()

    @pl.when(jnp.logical_and(s < NUM_SUBCORES - 1, cc == 0))
    def _():
        pltpu.sync_copy(counts_sh.at[pl.ds(off, CHUNK)],
                        out0_hbm.at[pl.ds(off, CHUNK)])

    @pl.when(jnp.logical_and(s == NUM_SUBCORES - 1, cc == 0))
    def _():
        pltpu.sync_copy(counts_sh.at[pl.ds(off, LAST_CHUNK)],
                        out0_hbm.at[pl.ds(off, LAST_CHUNK)])

    @pl.when(jnp.logical_and(s < NUM_SUBCORES - 1, cc == 1))
    def _():
        pltpu.sync_copy(counts_sh.at[pl.ds(off, CHUNK)],
                        out1_hbm.at[pl.ds(off, CHUNK)])

    @pl.when(jnp.logical_and(s == NUM_SUBCORES - 1, cc == 1))
    def _():
        pltpu.sync_copy(counts_sh.at[pl.ds(off, LAST_CHUNK)],
                        out1_hbm.at[pl.ds(off, LAST_CHUNK)])


@functools.cache
def _count_kernel():
    return pl.kernel(
        _count_body,
        out_type=[
            jax.ShapeDtypeStruct((VOCAB,), jnp.float32),
            jax.ShapeDtypeStruct((VOCAB,), jnp.float32),
        ],
        mesh=plsc.VectorSubcoreMesh(
            core_axis_name="c", subcore_axis_name="s",
            num_cores=NUM_CORES, num_subcores=NUM_SUBCORES,
        ),
        scratch_types=[
            pltpu.VMEM((ROWS_PER_TILE, CB), jnp.int32),
            pltpu.VMEM((CB,), jnp.float32),
            pltpu.VMEM((CHUNK,), jnp.float32),
            pltpu.VMEM_SHARED((VOCAB,), jnp.float32),
            pltpu.SemaphoreType.DMA,
        ],
        compiler_params=pltpu.CompilerParams(use_tc_tiling_on_sc=False),
    )


BV = 20480
NB = (VOCAB + BV - 1) // BV


def _fused_body(c0_ref, c1_ref, et_ref, wt_ref, b_ref, o_ref, acc_ref):
    p = pl.program_id(0)
    j = pl.program_id(1)

    @pl.when(jnp.logical_and(p == 0, j == 0))
    def _():
        acc_ref[...] = jnp.zeros_like(acc_ref)

    @pl.when(p == 0)
    def _():
        col = j * BV + lax.broadcasted_iota(jnp.int32, (1, BV), 1)
        valid = col < VOCAB
        c = jnp.where(valid, (c0_ref[...] + c1_ref[...])[None, :], 0.0)
        et = jnp.where(jnp.broadcast_to(valid, (EMBED, BV)), et_ref[...], 0.0)
        acc_ref[...] += lax.dot_general(
            c, et, (((1,), (1,)), ((), ())),
            preferred_element_type=jnp.float32,
        )

    @pl.when(p == 1)
    def _():
        res = lax.dot_general(
            acc_ref[...], wt_ref[...], (((1,), (0,)), ((), ())),
            preferred_element_type=jnp.float32,
        )
        o_ref[...] = res[0] + b_ref[...]


def _fused(c0, c1, embT, WT, b):
    return pl.pallas_call(
        _fused_body,
        grid=(2, NB),
        in_specs=[
            pl.BlockSpec((BV,), lambda p, j: (j * (1 - p) + (NB - 1) * p,)),
            pl.BlockSpec((BV,), lambda p, j: (j * (1 - p) + (NB - 1) * p,)),
            pl.BlockSpec((EMBED, BV),
                         lambda p, j: (0, j * (1 - p) + (NB - 1) * p)),
            pl.BlockSpec((EMBED, BV), lambda p, j: (0, j * p)),
            pl.BlockSpec((BV,), lambda p, j: (j * p,)),
        ],
        out_specs=pl.BlockSpec((BV,), lambda p, j: (j * p,)),
        out_shape=jax.ShapeDtypeStruct((VOCAB,), jnp.float32),
        scratch_shapes=[pltpu.VMEM((1, EMBED), jnp.float32)],
    )(c0, c1, embT, WT, b)


def kernel(inputs, emb, W, b):
    idx2 = inputs.astype(jnp.int32).reshape(NROWS, CB)
    c0, c1 = _count_kernel()(idx2)
    return _fused(c0, c1, emb.T, W.T, b)


# speedup vs baseline: 3.3537x; 1.0129x over previous
import functools

import jax
import jax.numpy as jnp
from jax import lax
from jax.experimental import pallas as pl
from jax.experimental.pallas import tpu as pltpu
from jax.experimental.pallas import tpu_sc as plsc

VOCAB = 100000
EMBED = 64
N = 16384

NUM_CORES = 2
NUM_SUBCORES = 16
LANES = 16

CB = 128
NROWS = N // CB
ROWS_PER_TILE = NROWS // (NUM_CORES * NUM_SUBCORES)

CHUNK = 6256
LAST_CHUNK = VOCAB - CHUNK * (NUM_SUBCORES - 1)


def _count_body(idx_hbm, out0_hbm, out1_hbm, idx_v, ones_v, zeros_v, counts_sh,
                sem):
    cc = lax.axis_index("c")
    s = lax.axis_index("s")

    row0 = cc * (NROWS // NUM_CORES) + s * ROWS_PER_TILE
    idx_cps = [
        pltpu.async_copy(idx_hbm.at[row0 + j], idx_v.at[j], sem)
        for j in range(ROWS_PER_TILE)
    ]

    one = jnp.full((LANES,), 1.0, jnp.float32)
    for k in range(CB // LANES):
        ones_v[pl.ds(k * LANES, LANES)] = one

    zero = jnp.zeros((LANES,), jnp.float32)

    def zfill(i, carry):
        zeros_v[pl.ds(pl.multiple_of(i * LANES, LANES), LANES)] = zero
        return carry

    lax.fori_loop(0, CHUNK // LANES, zfill, 0, unroll=8)

    off = pl.multiple_of(s * CHUNK, 8)

    @pl.when(s < NUM_SUBCORES - 1)
    def _():
        pltpu.sync_copy(zeros_v, counts_sh.at[pl.ds(off, CHUNK)])

    @pl.when(s == NUM_SUBCORES - 1)
    def _():
        pltpu.sync_copy(zeros_v.at[pl.ds(0, LAST_CHUNK)],
                        counts_sh.at[pl.ds(off, LAST_CHUNK)])

    for cp in idx_cps:
        cp.wait()
    plsc.subcore_barrier()

    for j in range(ROWS_PER_TILE):
        pltpu.sync_copy(ones_v, counts_sh.at[idx_v.at[j]], add=True)

    plsc.subcore_barrier()

    @pl.when(jnp.logical_and(s < NUM_SUBCORES - 1, cc == 0))
    def _():
        pltpu.sync_copy(counts_sh.at[pl.ds(off, CHUNK)],
                        out0_hbm.at[pl.ds(off, CHUNK)])

    @pl.when(jnp.logical_and(s == NUM_SUBCORES - 1, cc == 0))
    def _():
        pltpu.sync_copy(counts_sh.at[pl.ds(off, LAST_CHUNK)],
                        out0_hbm.at[pl.ds(off, LAST_CHUNK)])

    @pl.when(jnp.logical_and(s < NUM_SUBCORES - 1, cc == 1))
    def _():
        pltpu.sync_copy(counts_sh.at[pl.ds(off, CHUNK)],
                        out1_hbm.at[pl.ds(off, CHUNK)])

    @pl.when(jnp.logical_and(s == NUM_SUBCORES - 1, cc == 1))
    def _():
        pltpu.sync_copy(counts_sh.at[pl.ds(off, LAST_CHUNK)],
                        out1_hbm.at[pl.ds(off, LAST_CHUNK)])


@functools.cache
def _count_kernel():
    return pl.kernel(
        _count_body,
        out_type=[
            jax.ShapeDtypeStruct((VOCAB,), jnp.float32),
            jax.ShapeDtypeStruct((VOCAB,), jnp.float32),
        ],
        mesh=plsc.VectorSubcoreMesh(
            core_axis_name="c", subcore_axis_name="s",
            num_cores=NUM_CORES, num_subcores=NUM_SUBCORES,
        ),
        scratch_types=[
            pltpu.VMEM((ROWS_PER_TILE, CB), jnp.int32),
            pltpu.VMEM((CB,), jnp.float32),
            pltpu.VMEM((CHUNK,), jnp.float32),
            pltpu.VMEM_SHARED((VOCAB,), jnp.float32),
            pltpu.SemaphoreType.DMA,
        ],
        compiler_params=pltpu.CompilerParams(use_tc_tiling_on_sc=False),
    )


BV = 25600
NB = (VOCAB + BV - 1) // BV


def _fused_body(c0_ref, c1_ref, et_ref, wt_ref, b_ref, o_ref, acc_ref):
    p = pl.program_id(0)
    j = pl.program_id(1)

    @pl.when(jnp.logical_and(p == 0, j == 0))
    def _():
        acc_ref[...] = jnp.zeros_like(acc_ref)

    @pl.when(p == 0)
    def _():
        col = j * BV + lax.broadcasted_iota(jnp.int32, (1, BV), 1)
        valid = col < VOCAB
        c = jnp.where(valid, (c0_ref[...] + c1_ref[...])[None, :], 0.0)
        et = jnp.where(jnp.broadcast_to(valid, (EMBED, BV)), et_ref[...], 0.0)
        acc_ref[...] += lax.dot_general(
            c, et, (((1,), (1,)), ((), ())),
            preferred_element_type=jnp.float32,
        )

    @pl.when(p == 1)
    def _():
        res = lax.dot_general(
            acc_ref[...], wt_ref[...], (((1,), (0,)), ((), ())),
            preferred_element_type=jnp.float32,
        )
        o_ref[...] = res[0] + b_ref[...]


def _fused(c0, c1, embT, WT, b):
    return pl.pallas_call(
        _fused_body,
        grid=(2, NB),
        in_specs=[
            pl.BlockSpec((BV,), lambda p, j: (j * (1 - p) + (NB - 1) * p,)),
            pl.BlockSpec((BV,), lambda p, j: (j * (1 - p) + (NB - 1) * p,)),
            pl.BlockSpec((EMBED, BV),
                         lambda p, j: (0, j * (1 - p) + (NB - 1) * p)),
            pl.BlockSpec((EMBED, BV), lambda p, j: (0, j * p)),
            pl.BlockSpec((BV,), lambda p, j: (j * p,)),
        ],
        out_specs=pl.BlockSpec((BV,), lambda p, j: (j * p,)),
        out_shape=jax.ShapeDtypeStruct((VOCAB,), jnp.float32),
        scratch_shapes=[pltpu.VMEM((1, EMBED), jnp.float32)],
    )(c0, c1, embT, WT, b)


def kernel(inputs, emb, W, b):
    idx2 = inputs.astype(jnp.int32).reshape(NROWS, CB)
    c0, c1 = _count_kernel()(idx2)
    return _fused(c0, c1, emb.T, W.T, b)


# speedup vs baseline: 3.3617x; 1.0024x over previous
import functools

import jax
import jax.numpy as jnp
from jax import lax
from jax.experimental import pallas as pl
from jax.experimental.pallas import tpu as pltpu
from jax.experimental.pallas import tpu_sc as plsc

VOCAB = 100000
EMBED = 64
N = 16384

NUM_CORES = 2
NUM_SUBCORES = 16
LANES = 16

CB = 128
NROWS = N // CB
ROWS_PER_TILE = NROWS // (NUM_CORES * NUM_SUBCORES)

CHUNK = 6256
LAST_CHUNK = VOCAB - CHUNK * (NUM_SUBCORES - 1)


def _count_body(idx_hbm, out0_hbm, out1_hbm, idx_v, ones_v, zeros_v, counts_sh,
                sem):
    cc = lax.axis_index("c")
    s = lax.axis_index("s")

    row0 = cc * (NROWS // NUM_CORES) + s * ROWS_PER_TILE
    idx_cps = [
        pltpu.async_copy(idx_hbm.at[row0 + j], idx_v.at[j], sem)
        for j in range(ROWS_PER_TILE)
    ]

    one = jnp.full((LANES,), 1.0, jnp.float32)
    for k in range(CB // LANES):
        ones_v[pl.ds(k * LANES, LANES)] = one

    zero = jnp.zeros((LANES,), jnp.float32)

    def zfill(i, carry):
        zeros_v[pl.ds(pl.multiple_of(i * LANES, LANES), LANES)] = zero
        return carry

    lax.fori_loop(0, CHUNK // LANES, zfill, 0, unroll=8)

    off = pl.multiple_of(s * CHUNK, 8)

    @pl.when(s < NUM_SUBCORES - 1)
    def _():
        pltpu.sync_copy(zeros_v, counts_sh.at[pl.ds(off, CHUNK)])

    @pl.when(s == NUM_SUBCORES - 1)
    def _():
        pltpu.sync_copy(zeros_v.at[pl.ds(0, LAST_CHUNK)],
                        counts_sh.at[pl.ds(off, LAST_CHUNK)])

    for cp in idx_cps:
        cp.wait()
    plsc.subcore_barrier()

    for j in range(ROWS_PER_TILE):
        pltpu.sync_copy(ones_v, counts_sh.at[idx_v.at[j]], add=True)

    plsc.subcore_barrier()

    @pl.when(jnp.logical_and(s < NUM_SUBCORES - 1, cc == 0))
    def _():
        pltpu.sync_copy(counts_sh.at[pl.ds(off, CHUNK)],
                        out0_hbm.at[pl.ds(off, CHUNK)])

    @pl.when(jnp.logical_and(s == NUM_SUBCORES - 1, cc == 0))
    def _():
        pltpu.sync_copy(counts_sh.at[pl.ds(off, LAST_CHUNK)],
                        out0_hbm.at[pl.ds(off, LAST_CHUNK)])

    @pl.when(jnp.logical_and(s < NUM_SUBCORES - 1, cc == 1))
    def _():
        pltpu.sync_copy(counts_sh.at[pl.ds(off, CHUNK)],
                        out1_hbm.at[pl.ds(off, CHUNK)])

    @pl.when(jnp.logical_and(s == NUM_SUBCORES - 1, cc == 1))
    def _():
        pltpu.sync_copy(counts_sh.at[pl.ds(off, LAST_CHUNK)],
                        out1_hbm.at[pl.ds(off, LAST_CHUNK)])


@functools.cache
def _count_kernel():
    return pl.kernel(
        _count_body,
        out_type=[
            jax.ShapeDtypeStruct((VOCAB,), jnp.float32),
            jax.ShapeDtypeStruct((VOCAB,), jnp.float32),
        ],
        mesh=plsc.VectorSubcoreMesh(
            core_axis_name="c", subcore_axis_name="s",
            num_cores=NUM_CORES, num_subcores=NUM_SUBCORES,
        ),
        scratch_types=[
            pltpu.VMEM((ROWS_PER_TILE, CB), jnp.int32),
            pltpu.VMEM((CB,), jnp.float32),
            pltpu.VMEM((CHUNK,), jnp.float32),
            pltpu.VMEM_SHARED((VOCAB,), jnp.float32),
            pltpu.SemaphoreType.DMA,
        ],
        compiler_params=pltpu.CompilerParams(use_tc_tiling_on_sc=False),
    )


BV = 34816
NB = (VOCAB + BV - 1) // BV


def _fused_body(c0_ref, c1_ref, et_ref, wt_ref, b_ref, o_ref, acc_ref):
    p = pl.program_id(0)
    j = pl.program_id(1)

    @pl.when(jnp.logical_and(p == 0, j == 0))
    def _():
        acc_ref[...] = jnp.zeros_like(acc_ref)

    @pl.when(p == 0)
    def _():
        col = j * BV + lax.broadcasted_iota(jnp.int32, (1, BV), 1)
        valid = col < VOCAB
        c = jnp.where(valid, (c0_ref[...] + c1_ref[...])[None, :], 0.0)
        et = jnp.where(jnp.broadcast_to(valid, (EMBED, BV)), et_ref[...], 0.0)
        acc_ref[...] += lax.dot_general(
            c, et, (((1,), (1,)), ((), ())),
            preferred_element_type=jnp.float32,
        )

    @pl.when(p == 1)
    def _():
        res = lax.dot_general(
            acc_ref[...], wt_ref[...], (((1,), (0,)), ((), ())),
            preferred_element_type=jnp.float32,
        )
        o_ref[...] = res[0] + b_ref[...]


def _fused(c0, c1, embT, WT, b):
    return pl.pallas_call(
        _fused_body,
        grid=(2, NB),
        in_specs=[
            pl.BlockSpec((BV,), lambda p, j: (j * (1 - p) + (NB - 1) * p,)),
            pl.BlockSpec((BV,), lambda p, j: (j * (1 - p) + (NB - 1) * p,)),
            pl.BlockSpec((EMBED, BV),
                         lambda p, j: (0, j * (1 - p) + (NB - 1) * p)),
            pl.BlockSpec((EMBED, BV), lambda p, j: (0, j * p)),
            pl.BlockSpec((BV,), lambda p, j: (j * p,)),
        ],
        out_specs=pl.BlockSpec((BV,), lambda p, j: (j * p,)),
        out_shape=jax.ShapeDtypeStruct((VOCAB,), jnp.float32),
        scratch_shapes=[pltpu.VMEM((1, EMBED), jnp.float32)],
    )(c0, c1, embT, WT, b)


def kernel(inputs, emb, W, b):
    idx2 = inputs.astype(jnp.int32).reshape(NROWS, CB)
    c0, c1 = _count_kernel()(idx2)
    return _fused(c0, c1, emb.T, W.T, b)
